# baseline (speedup 1.0000x reference)
import functools

import jax
import jax.numpy as jnp
from jax import lax
from jax.experimental import pallas as pl
from jax.experimental.pallas import tpu as pltpu
from jax.experimental.pallas import tpu_sc as plsc

B = 1024
L = 50
C = 1000
ROW = C * L
LP = 64
NW = 32
RPW = B // NW


def _onehot_body(t_hbm, out_hbm, t_v, row_v):
    wid = lax.axis_index("s") * 2 + lax.axis_index("c")
    base = wid * RPW

    pltpu.sync_copy(t_hbm.at[pl.ds(base, RPW)], t_v)

    zeros16 = jnp.zeros((16,), jnp.float32)

    def _ms(k, carry):
        row_v[pl.ds(k * 16, 16)] = zeros16
        return carry

    lax.fori_loop(0, ROW // 16, _ms, 0)

    ones16 = jnp.ones((16,), jnp.float32)
    iota16 = lax.iota(jnp.int32, 16)
    tail_mask = iota16 < (L - 48)

    def _positions(i, j):
        tl = t_v[i, pl.ds(j * 16, 16)]
        return tl * L + (iota16 + j * 16)

    for i in range(RPW):
        for j in range(LP // 16):
            p = _positions(i, j)
            if (j + 1) * 16 <= L:
                plsc.store_scatter(row_v, [p], ones16)
            else:
                plsc.store_scatter(row_v, [p], ones16, mask=tail_mask)
        pltpu.sync_copy(row_v, out_hbm.at[base + i])
        for j in range(LP // 16):
            p = _positions(i, j)
            if (j + 1) * 16 <= L:
                plsc.store_scatter(row_v, [p], zeros16)
            else:
                plsc.store_scatter(row_v, [p], zeros16, mask=tail_mask)


_onehot_sc = functools.partial(
    pl.kernel,
    mesh=plsc.VectorSubcoreMesh(core_axis_name="c", subcore_axis_name="s"),
    out_type=jax.ShapeDtypeStruct((B, ROW), jnp.float32),
    scratch_types=[
        pltpu.VMEM((RPW, LP), jnp.int32),
        pltpu.VMEM((ROW,), jnp.float32),
    ],
    compiler_params=pltpu.CompilerParams(needs_layout_passes=False),
)(_onehot_body)


def kernel(t, ones):
    del ones
    t_pad = jnp.pad(t.astype(jnp.int32), ((0, 0), (0, LP - L)))
    out2d = _onehot_sc(t_pad)
    return out2d.reshape(B, C, L)


# speedup vs baseline: 2.1085x; 2.1085x over previous
import functools

import jax
import jax.numpy as jnp
from jax import lax
from jax.experimental import pallas as pl
from jax.experimental.pallas import tpu as pltpu
from jax.experimental.pallas import tpu_sc as plsc

B = 1024
L = 50
C = 1000
LP = 64
NW = 32
RPW = B // NW


T_CHUNK = 8


def _onehot_body(t_hbm, out_hbm, t_v, img_v):
    wid = lax.axis_index("s") * 2 + lax.axis_index("c")
    base = wid * RPW

    zeros16 = jnp.zeros((16,), jnp.float32)

    def _ms(r, carry):
        img_v[r, pl.ds(0, 16)] = zeros16
        img_v[r, pl.ds(16, 16)] = zeros16
        img_v[r, pl.ds(32, 16)] = zeros16
        img_v[r, pl.ds(L - 16, 16)] = zeros16
        return carry

    lax.fori_loop(0, C, _ms, 0)

    ones16 = jnp.ones((16,), jnp.float32)
    iota16 = lax.iota(jnp.int32, 16)
    tail_mask = iota16 < (L - 48)

    for i in range(RPW):
        if i % T_CHUNK == 0:
            pltpu.sync_copy(t_hbm.at[pl.ds(base + i, T_CHUNK)], t_v)
        for j in range(LP // 16):
            cls = t_v[i % T_CHUNK, pl.ds(j * 16, 16)]
            pos = iota16 + j * 16
            if (j + 1) * 16 <= L:
                plsc.store_scatter(img_v, [cls, pos], ones16)
            else:
                plsc.store_scatter(img_v, [cls, pos], ones16, mask=tail_mask)
        pltpu.sync_copy(img_v, out_hbm.at[base + i])
        for j in range(LP // 16):
            cls = t_v[i % T_CHUNK, pl.ds(j * 16, 16)]
            pos = iota16 + j * 16
            if (j + 1) * 16 <= L:
                plsc.store_scatter(img_v, [cls, pos], zeros16)
            else:
                plsc.store_scatter(img_v, [cls, pos], zeros16, mask=tail_mask)


_onehot_sc = functools.partial(
    pl.kernel,
    mesh=plsc.VectorSubcoreMesh(core_axis_name="c", subcore_axis_name="s"),
    out_type=jax.ShapeDtypeStruct((B, C, L), jnp.float32),
    scratch_types=[
        pltpu.VMEM((T_CHUNK, LP), jnp.int32),
        pltpu.VMEM((C, L), jnp.float32),
    ],
    compiler_params=pltpu.CompilerParams(needs_layout_passes=False),
)(_onehot_body)


def kernel(t, ones):
    del ones
    t_pad = jnp.pad(t.astype(jnp.int32), ((0, 0), (0, LP - L)))
    return _onehot_sc(t_pad)


# speedup vs baseline: 8.3460x; 3.9582x over previous
import functools

import jax
import jax.numpy as jnp
from jax import lax
from jax.experimental import pallas as pl
from jax.experimental.pallas import tpu as pltpu
from jax.experimental.pallas import tpu_sc as plsc

B = 1024
L = 50
C = 1000
NW = 32
CROWS = 40
CPP = C // CROWS
NCHUNK = L * CPP
NC_LO = NCHUNK // NW
NC_REM = NCHUNK - NC_LO * NW


def _onehot_body(t_hbm, y_hbm, t_v, buf_v):
    wid = lax.axis_index("s") * 2 + lax.axis_index("c")
    nc = jnp.where(wid < NC_REM, NC_LO + 1, NC_LO)
    qc0 = wid * NC_LO + lax.min(wid, NC_REM)

    pltpu.sync_copy(t_hbm, t_v)

    zeros16 = jnp.zeros((16,), jnp.float32)
    ones16 = jnp.ones((16,), jnp.float32)
    iota16 = lax.iota(jnp.int32, 16)

    def _ms(i, carry):
        buf_v[i // (B // 16), pl.ds((i % (B // 16)) * 16, 16)] = zeros16
        return carry

    lax.fori_loop(0, CROWS * (B // 16), _ms, 0)

    def _scatter(l, c_lo, c_hi, val16):
        def _v(v, carry):
            tl = t_v[l, pl.ds(v * 16, 16)]
            m = (tl >= c_lo) & (tl < c_hi)
            plsc.store_scatter(buf_v, [tl - c_lo, iota16 + v * 16], val16,
                               mask=m)
            return carry

        lax.fori_loop(0, B // 16, _v, 0)

    def _chunk(k, carry):
        qc = qc0 + k
        l = qc // CPP
        c_lo = (qc - l * CPP) * CROWS
        c_hi = c_lo + CROWS
        _scatter(l, c_lo, c_hi, ones16)
        pltpu.sync_copy(buf_v, y_hbm.at[l, pl.ds(c_lo, CROWS)])
        _scatter(l, c_lo, c_hi, zeros16)
        return carry

    lax.fori_loop(0, nc, _chunk, 0)


_onehot_sc = functools.partial(
    pl.kernel,
    mesh=plsc.VectorSubcoreMesh(core_axis_name="c", subcore_axis_name="s"),
    out_type=jax.ShapeDtypeStruct((L, C, B), jnp.float32),
    scratch_types=[
        pltpu.VMEM((L, B), jnp.int32),
        pltpu.VMEM((CROWS, B), jnp.float32),
    ],
    compiler_params=pltpu.CompilerParams(needs_layout_passes=False),
)(_onehot_body)


def kernel(t, ones):
    del ones
    y = _onehot_sc(jnp.transpose(t.astype(jnp.int32)))
    return jnp.transpose(y, (2, 1, 0))


# speedup vs baseline: 12.2791x; 1.4713x over previous
import functools

import jax
import jax.numpy as jnp
from jax import lax
from jax.experimental import pallas as pl
from jax.experimental.pallas import tpu as pltpu
from jax.experimental.pallas import tpu_sc as plsc

B = 1024
L = 50
C = 1000
LP = 64
NW = 32
CROWS = 40
CPP = C // CROWS
NCHUNK = L * CPP
NC_LO = NCHUNK // NW
NC_REM = NCHUNK - NC_LO * NW
NC_MAX = NC_LO + 1
TW = 16


def _onehot_body(t_hbm, y_hbm, t_v, buf0, buf1, sem0, sem1):
    wid = lax.axis_index("s") * 2 + lax.axis_index("c")
    nc = jnp.where(wid < NC_REM, NC_LO + 1, NC_LO)
    qc0 = wid * NC_LO + lax.min(wid, NC_REM)

    lw0 = (qc0 // CPP) // 8 * 8
    pltpu.sync_copy(t_hbm.at[pl.ds(lw0, TW)], t_v)

    zeros16 = jnp.zeros((16,), jnp.float32)
    ones16 = jnp.ones((16,), jnp.float32)
    iota16 = lax.iota(jnp.int32, 16)
    bufs = (buf0, buf1)
    sems = (sem0, sem1)

    def _ms(i, carry):
        r, v = i // (B // 16), i % (B // 16)
        buf0[r, pl.ds(v * 16, 16)] = zeros16
        buf1[r, pl.ds(v * 16, 16)] = zeros16
        return carry

    lax.fori_loop(0, CROWS * (B // 16), _ms, 0)

    def _scatter(buf, qc, val16):
        l = qc // CPP
        c_lo = (qc - l * CPP) * CROWS

        def _v(v, carry):
            tl = t_v[l - lw0, pl.ds(v * 16, 16)]
            m = (tl >= c_lo) & (tl < c_lo + CROWS)
            plsc.store_scatter(buf, [tl - c_lo, iota16 + v * 16], val16,
                               mask=m)
            return carry

        lax.fori_loop(0, B // 16, _v, 0)
        return l, c_lo

    def _pair(k2, carry):
        for half in range(2):
            k = k2 * 2 + half
            buf, sem = bufs[half], sems[half]

            @pl.when(k < nc)
            def _():
                @pl.when(k >= 2)
                def _():
                    pltpu.make_async_copy(
                        buf, y_hbm.at[0, pl.ds(0, CROWS)], sem).wait()
                    _scatter(buf, qc0 + k - 2, zeros16)

                l, c_lo = _scatter(buf, qc0 + k, ones16)
                pltpu.async_copy(buf, y_hbm.at[l, pl.ds(c_lo, CROWS)], sem)

        return carry

    lax.fori_loop(0, (NC_MAX + 1) // 2, _pair, 0)

    pltpu.make_async_copy(buf0, y_hbm.at[0, pl.ds(0, CROWS)], sem0).wait()
    pltpu.make_async_copy(buf1, y_hbm.at[0, pl.ds(0, CROWS)], sem1).wait()


_onehot_sc = functools.partial(
    pl.kernel,
    mesh=plsc.VectorSubcoreMesh(core_axis_name="c", subcore_axis_name="s"),
    out_type=jax.ShapeDtypeStruct((L, C, B), jnp.float32),
    scratch_types=[
        pltpu.VMEM((TW, B), jnp.int32),
        pltpu.VMEM((CROWS, B), jnp.float32),
        pltpu.VMEM((CROWS, B), jnp.float32),
        pltpu.SemaphoreType.DMA,
        pltpu.SemaphoreType.DMA,
    ],
    compiler_params=pltpu.CompilerParams(needs_layout_passes=False),
)(_onehot_body)


def kernel(t, ones):
    del ones
    t_pad = jnp.pad(jnp.transpose(t.astype(jnp.int32)), ((0, LP - L), (0, 0)))
    return jnp.transpose(_onehot_sc(t_pad), (2, 1, 0))


# speedup vs baseline: 12.4516x; 1.0140x over previous
import functools

import jax
import jax.numpy as jnp
from jax import lax
from jax.experimental import pallas as pl
from jax.experimental.pallas import tpu as pltpu
from jax.experimental.pallas import tpu_sc as plsc

B = 1024
L = 50
C = 1000
LP = 64
NW = 32
CROWS = 40
CPP = C // CROWS
NCHUNK = L * CPP
NC_LO = NCHUNK // NW
NC_REM = NCHUNK - NC_LO * NW
NC_MAX = NC_LO + 1
TW = 16


def _onehot_body(t_hbm, z_hbm, y_hbm, t_v, buf0, buf1, sem0, sem1):
    wid = lax.axis_index("s") * 2 + lax.axis_index("c")
    nc = jnp.where(wid < NC_REM, NC_LO + 1, NC_LO)
    qc0 = wid * NC_LO + lax.min(wid, NC_REM)

    lw0 = (qc0 // CPP) // 8 * 8
    h_t = pltpu.async_copy(t_hbm.at[pl.ds(lw0, TW)], t_v, sem0)
    h_z0 = pltpu.async_copy(z_hbm, buf0, sem1)
    h_z1 = pltpu.async_copy(z_hbm, buf1, sem1)
    h_t.wait()
    h_z0.wait()
    h_z1.wait()

    zeros16 = jnp.zeros((16,), jnp.float32)
    ones16 = jnp.ones((16,), jnp.float32)
    iota16 = lax.iota(jnp.int32, 16)
    bufs = (buf0, buf1)
    sems = (sem0, sem1)

    def _scatter(buf, qc, val16):
        l = qc // CPP
        c_lo = (qc - l * CPP) * CROWS

        def _v(v4, carry):
            for u in range(4):
                v = v4 * 4 + u
                tl = t_v[l - lw0, pl.ds(v * 16, 16)]
                m = (tl >= c_lo) & (tl < c_lo + CROWS)
                plsc.store_scatter(buf, [tl - c_lo, iota16 + v * 16], val16,
                                   mask=m)
            return carry

        lax.fori_loop(0, B // 64, _v, 0)
        return l, c_lo

    def _pair(k2, carry):
        for half in range(2):
            k = k2 * 2 + half
            buf, sem = bufs[half], sems[half]

            @pl.when(k < nc)
            def _():
                @pl.when(k >= 2)
                def _():
                    pltpu.make_async_copy(
                        buf, y_hbm.at[0, pl.ds(0, CROWS)], sem).wait()
                    _scatter(buf, qc0 + k - 2, zeros16)

                l, c_lo = _scatter(buf, qc0 + k, ones16)
                pltpu.async_copy(buf, y_hbm.at[l, pl.ds(c_lo, CROWS)], sem)

        return carry

    lax.fori_loop(0, (NC_MAX + 1) // 2, _pair, 0)

    pltpu.make_async_copy(buf0, y_hbm.at[0, pl.ds(0, CROWS)], sem0).wait()
    pltpu.make_async_copy(buf1, y_hbm.at[0, pl.ds(0, CROWS)], sem1).wait()


_onehot_sc = functools.partial(
    pl.kernel,
    mesh=plsc.VectorSubcoreMesh(core_axis_name="c", subcore_axis_name="s"),
    out_type=jax.ShapeDtypeStruct((L, C, B), jnp.float32),
    scratch_types=[
        pltpu.VMEM((TW, B), jnp.int32),
        pltpu.VMEM((CROWS, B), jnp.float32),
        pltpu.VMEM((CROWS, B), jnp.float32),
        pltpu.SemaphoreType.DMA,
        pltpu.SemaphoreType.DMA,
    ],
    compiler_params=pltpu.CompilerParams(needs_layout_passes=False),
)(_onehot_body)


def kernel(t, ones):
    del ones
    t_pad = jnp.pad(jnp.transpose(t.astype(jnp.int32)), ((0, LP - L), (0, 0)))
    z = jnp.zeros((CROWS, B), jnp.float32)
    return jnp.transpose(_onehot_sc(t_pad, z), (2, 1, 0))


# speedup vs baseline: 12.4623x; 1.0009x over previous
import functools

import jax
import jax.numpy as jnp
from jax import lax
from jax.experimental import pallas as pl
from jax.experimental.pallas import tpu as pltpu
from jax.experimental.pallas import tpu_sc as plsc

B = 1024
L = 50
C = 1000
LP = 64
NW = 32
CROWS = 40
CPP = C // CROWS
NCHUNK = L * CPP
NC_LO = NCHUNK // NW
NC_REM = NCHUNK - NC_LO * NW
NC_MAX = NC_LO + 1
TW = 16


def _onehot_body(t_hbm, z_hbm, y_hbm, t_v, buf0, buf1, sem0, sem1):
    wid = lax.axis_index("s") * 2 + lax.axis_index("c")
    nc = jnp.where(wid < NC_REM, NC_LO + 1, NC_LO)
    qc0 = wid * NC_LO + lax.min(wid, NC_REM)

    lw0 = (qc0 // CPP) // 8 * 8
    h_t = pltpu.async_copy(t_hbm.at[pl.ds(lw0, TW)], t_v, sem0)
    h_z0 = pltpu.async_copy(z_hbm, buf0, sem1)
    h_z1 = pltpu.async_copy(z_hbm, buf1, sem1)
    h_t.wait()
    h_z0.wait()
    h_z1.wait()

    zeros16 = jnp.zeros((16,), jnp.float32)
    ones16 = jnp.ones((16,), jnp.float32)
    iota16 = lax.iota(jnp.int32, 16)
    bufs = (buf0, buf1)
    sems = (sem0, sem1)

    def _scatter(buf, qc, val16):
        l = qc // CPP
        c_lo = (qc - l * CPP) * CROWS

        def _v(v4, carry):
            for u in range(4):
                v = v4 * 4 + u
                tl = t_v[l - lw0, pl.ds(v * 16, 16)]
                m = (tl >= c_lo) & (tl < c_lo + CROWS)
                plsc.store_scatter(buf, [tl - c_lo, iota16 + v * 16], val16,
                                   mask=m)
            return carry

        lax.fori_loop(0, B // 64, _v, 0)
        return l, c_lo

    def _pair(k2, carry):
        for half in range(2):
            k = k2 * 2 + half
            buf, sem = bufs[half], sems[half]

            @pl.when(k < nc)
            def _():
                @pl.when(k >= 2)
                def _():
                    pltpu.make_async_copy(
                        buf, y_hbm.at[0, pl.ds(0, CROWS)], sem).wait()
                    _scatter(buf, qc0 + k - 2, zeros16)

                l, c_lo = _scatter(buf, qc0 + k, ones16)
                pltpu.async_copy(buf, y_hbm.at[l, pl.ds(c_lo, CROWS)], sem)

        return carry

    lax.fori_loop(0, (NC_MAX + 1) // 2, _pair, 0)

    pltpu.make_async_copy(buf0, y_hbm.at[0, pl.ds(0, CROWS)], sem0).wait()
    pltpu.make_async_copy(buf1, y_hbm.at[0, pl.ds(0, CROWS)], sem1).wait()


_onehot_sc = functools.partial(
    pl.kernel,
    mesh=plsc.VectorSubcoreMesh(core_axis_name="c", subcore_axis_name="s"),
    out_type=jax.ShapeDtypeStruct((L, C, B), jnp.float32),
    scratch_types=[
        pltpu.VMEM((TW, B), jnp.int32),
        pltpu.VMEM((CROWS, B), jnp.float32),
        pltpu.VMEM((CROWS, B), jnp.float32),
        pltpu.SemaphoreType.DMA,
        pltpu.SemaphoreType.DMA,
    ],
    compiler_params=pltpu.CompilerParams(
        needs_layout_passes=False, skip_device_barrier=True),
)(_onehot_body)


def kernel(t, ones):
    del ones
    t_pad = jnp.pad(jnp.transpose(t.astype(jnp.int32)), ((0, LP - L), (0, 0)))
    z = jnp.zeros((CROWS, B), jnp.float32)
    return jnp.transpose(_onehot_sc(t_pad, z), (2, 1, 0))
